# Initial kernel scaffold; baseline (speedup 1.0000x reference)
import jax
import jax.numpy as jnp
from jax.experimental import pallas as pl


def kernel(encoder_outs, mem_matrix):
    raise NotImplementedError("write your pallas kernel here")


# speedup vs baseline: 11.7782x; 11.7782x over previous
import functools

import jax
import jax.numpy as jnp
from jax import lax
from jax.experimental import pallas as pl
from jax.experimental.pallas import tpu as pltpu
from jax.experimental.pallas import tpu_sc as plsc

B = 4096
D = 128
M = 100000
K = 32

BB = 256
MB = 2048
NB = B // BB
NT = 49
CPT = MB // 128
C = NT * CPT

NEG1 = -1e30
NEG2 = -2e30


def _k1_body(e_ref, m_ref, sim_ref, cm_ref):
    mj = pl.program_id(1)
    e = e_ref[...]
    mt = m_ref[...]
    inv_na = lax.rsqrt(jnp.sum(e * e, axis=1, keepdims=True))
    inv_nb = lax.rsqrt(jnp.sum(mt * mt, axis=1))
    prod = lax.dot_general(e, mt, (((1,), (1,)), ((), ())),
                           preferred_element_type=jnp.float32)
    sim = prod * inv_na * inv_nb[None, :]
    gcol = mj * MB + lax.broadcasted_iota(jnp.int32, (BB, MB), 1)
    sim = jnp.where(gcol < M, sim, NEG1)
    sim3 = sim.reshape(BB, CPT, 128)
    sim_ref[...] = sim3[None, None]
    cm_ref[...] = jnp.max(sim3, axis=2)[None, None]


def _k2_body(cm_ref, sidx_ref, cid_ref, cm_scr):
    bi = pl.program_id(0)
    cm_scr[...] = cm_ref[0]
    iota = lax.broadcasted_iota(jnp.int32, (BB, C), 1)
    lanek = lax.broadcasted_iota(jnp.int32, (BB, K), 1)

    def body(k, cid_acc):
        cm = cm_scr[...]
        m = jnp.max(cm, axis=1, keepdims=True)
        sel = cm == m
        pos = jnp.min(jnp.where(sel, iota, jnp.int32(1 << 20)),
                      axis=1, keepdims=True)
        cm_scr[...] = jnp.where(iota == pos, NEG2, cm)
        return jnp.where(lanek == k, pos, cid_acc)

    cid = lax.fori_loop(0, K, body, jnp.zeros((BB, K), jnp.int32))
    brow = lax.broadcasted_iota(jnp.int32, (BB, K), 0)
    sidx = (bi * (NT * BB * CPT) + (cid >> 4) * (BB * CPT)
            + brow * CPT + (cid & 15))
    sidx_ref[...] = sidx[None]
    cid_ref[...] = cid[None]


def _k4_body(cand_ref, cid_ref, gidx_ref, w_ref, cand_scr, gmap_scr):
    cand = cand_ref[0]
    cid = cid_ref[0]
    gmap = (cid[:, :, None] * 128
            + lax.broadcasted_iota(jnp.int32, (BB, K, 128), 2))
    cand_scr[...] = cand.reshape(BB, K * 128)
    gmap_scr[...] = gmap.reshape(BB, K * 128)
    lanek = lax.broadcasted_iota(jnp.int32, (BB, K), 1)

    def body(k, acc):
        v_acc, g_acc = acc
        cv = cand_scr[...]
        gm = gmap_scr[...]
        m = jnp.max(cv, axis=1, keepdims=True)
        sel = cv == m
        gi = jnp.min(jnp.where(sel, gm, jnp.int32(1 << 30)),
                     axis=1, keepdims=True)
        cand_scr[...] = jnp.where(sel & (gm == gi), NEG2, cv)
        v_acc = jnp.where(lanek == k, m, v_acc)
        g_acc = jnp.where(lanek == k, gi, g_acc)
        return (v_acc, g_acc)

    v, gidx = lax.fori_loop(
        0, K, body,
        (jnp.zeros((BB, K), jnp.float32), jnp.zeros((BB, K), jnp.int32)))
    r = jnp.maximum(v, 0.0)
    ex = jnp.exp(r - jnp.max(r, axis=1, keepdims=True))
    w = ex / jnp.sum(ex, axis=1, keepdims=True)
    gidx_ref[...] = gidx[None]
    w_ref[...] = w[None]


def _k6_body(g_ref, w_ref, o_ref):
    g = g_ref[0]
    w = w_ref[0]
    acc = g[:, 0, :] * w[:, 0][:, None]
    for k in range(1, K):
        acc = acc + g[:, k, :] * w[:, k][:, None]
    o_ref[...] = acc


def _sc_gather_rows(table, idx):
    n = idx.shape[0]
    per_w = n // 32
    steps = per_w // 128
    mesh = plsc.VectorSubcoreMesh(core_axis_name="c", subcore_axis_name="s")

    @functools.partial(
        pl.kernel,
        out_type=jax.ShapeDtypeStruct((n, 128), jnp.float32),
        mesh=mesh,
        scratch_types=[
            pltpu.VMEM((128,), jnp.int32),
            pltpu.VMEM((128, 128), jnp.float32),
            pltpu.SemaphoreType.DMA,
        ],
    )
    def gather(table_hbm, idx_hbm, out_hbm, idx_v, rows_v, sem):
        wid = lax.axis_index("s") * 2 + lax.axis_index("c")
        base0 = pl.multiple_of(wid * per_w, 128)

        def body(j, carry):
            base = pl.multiple_of(base0 + j * 128, 128)
            pltpu.sync_copy(idx_hbm.at[pl.ds(base, 128)], idx_v)
            pltpu.async_copy(table_hbm.at[idx_v], rows_v, sem).wait()
            pltpu.sync_copy(rows_v, out_hbm.at[pl.ds(base, 128)])
            return carry

        lax.fori_loop(0, steps, body, 0)

    return gather(table, idx)


def kernel(encoder_outs, mem_matrix):
    f32 = jnp.float32
    sim, cm = pl.pallas_call(
        _k1_body,
        grid=(NB, NT),
        in_specs=[
            pl.BlockSpec((BB, D), lambda bi, mj: (bi, 0)),
            pl.BlockSpec((MB, D), lambda bi, mj: (mj, 0)),
        ],
        out_specs=[
            pl.BlockSpec((1, 1, BB, CPT, 128), lambda bi, mj: (bi, mj, 0, 0, 0)),
            pl.BlockSpec((1, 1, BB, CPT), lambda bi, mj: (bi, mj, 0, 0)),
        ],
        out_shape=[
            jax.ShapeDtypeStruct((NB, NT, BB, CPT, 128), f32),
            jax.ShapeDtypeStruct((NB, NT, BB, CPT), f32),
        ],
        compiler_params=pltpu.CompilerParams(
            dimension_semantics=("arbitrary", "arbitrary")),
    )(encoder_outs, mem_matrix)

    cmt = cm.transpose(0, 2, 1, 3).reshape(NB, BB, C)

    sidx, cid = pl.pallas_call(
        _k2_body,
        grid=(NB,),
        in_specs=[pl.BlockSpec((1, BB, C), lambda bi: (bi, 0, 0))],
        out_specs=[
            pl.BlockSpec((1, BB, K), lambda bi: (bi, 0, 0)),
            pl.BlockSpec((1, BB, K), lambda bi: (bi, 0, 0)),
        ],
        out_shape=[
            jax.ShapeDtypeStruct((NB, BB, K), jnp.int32),
            jax.ShapeDtypeStruct((NB, BB, K), jnp.int32),
        ],
        scratch_shapes=[pltpu.VMEM((BB, C), jnp.float32)],
    )(cmt)

    sim_flat = sim.reshape(NB * NT * BB * CPT, 128)
    cand_flat = _sc_gather_rows(sim_flat, sidx.reshape(-1))
    cand = cand_flat.reshape(NB, BB, K, 128)

    gidx, w = pl.pallas_call(
        _k4_body,
        grid=(NB,),
        in_specs=[
            pl.BlockSpec((1, BB, K, 128), lambda bi: (bi, 0, 0, 0)),
            pl.BlockSpec((1, BB, K), lambda bi: (bi, 0, 0)),
        ],
        out_specs=[
            pl.BlockSpec((1, BB, K), lambda bi: (bi, 0, 0)),
            pl.BlockSpec((1, BB, K), lambda bi: (bi, 0, 0)),
        ],
        out_shape=[
            jax.ShapeDtypeStruct((NB, BB, K), jnp.int32),
            jax.ShapeDtypeStruct((NB, BB, K), f32),
        ],
        scratch_shapes=[
            pltpu.VMEM((BB, K * 128), jnp.float32),
            pltpu.VMEM((BB, K * 128), jnp.int32),
        ],
    )(cand, cid)

    g_flat = _sc_gather_rows(mem_matrix, gidx.reshape(-1))
    g = g_flat.reshape(NB, BB, K, 128)

    out = pl.pallas_call(
        _k6_body,
        grid=(NB,),
        in_specs=[
            pl.BlockSpec((1, BB, K, 128), lambda bi: (bi, 0, 0, 0)),
            pl.BlockSpec((1, BB, K), lambda bi: (bi, 0, 0)),
        ],
        out_specs=pl.BlockSpec((BB, D), lambda bi: (bi, 0)),
        out_shape=jax.ShapeDtypeStruct((B, D), f32),
    )(g, w)
    return out


# speedup vs baseline: 11.8918x; 1.0096x over previous
import functools

import jax
import jax.numpy as jnp
from jax import lax
from jax.experimental import pallas as pl
from jax.experimental.pallas import tpu as pltpu
from jax.experimental.pallas import tpu_sc as plsc

B = 4096
D = 128
M = 100000
K = 32

BB = 256
MB = 2048
NB = B // BB
NT = 49
CPT = MB // 128
C = NT * CPT

NEG1 = -1e30
NEG2 = -2e30


def _nb_body(m_ref, nb_ref):
    mt = m_ref[...]
    nb_ref[...] = lax.rsqrt(jnp.sum(mt * mt, axis=1))[None, None]


def _k1_body(e_ref, m_ref, nb_ref, sim_ref, cm_ref):
    mj = pl.program_id(1)
    e = e_ref[...]
    mt = m_ref[...]
    inv_na = lax.rsqrt(jnp.sum(e * e, axis=1, keepdims=True))
    inv_nb = nb_ref[0, 0]
    prod = lax.dot_general(e, mt, (((1,), (1,)), ((), ())),
                           preferred_element_type=jnp.float32)
    sim = prod * inv_na * inv_nb[None, :]

    @pl.when(mj < NT - 1)
    def _full():
        sim3 = sim.reshape(BB, CPT, 128)
        sim_ref[...] = sim3[None, None]
        cm_ref[...] = jnp.max(sim3, axis=2)[None, None]

    @pl.when(mj == NT - 1)
    def _masked():
        gcol = mj * MB + lax.broadcasted_iota(jnp.int32, (BB, MB), 1)
        simm = jnp.where(gcol < M, sim, NEG1)
        sim3 = simm.reshape(BB, CPT, 128)
        sim_ref[...] = sim3[None, None]
        cm_ref[...] = jnp.max(sim3, axis=2)[None, None]


def _k2_body(cm_ref, sidx_ref, cid_ref, cm_scr):
    bi = pl.program_id(0)
    cm_scr[...] = cm_ref[0]
    iota = lax.broadcasted_iota(jnp.int32, (BB, C), 1)
    lanek = lax.broadcasted_iota(jnp.int32, (BB, K), 1)

    def body(k, cid_acc):
        cm = cm_scr[...]
        m = jnp.max(cm, axis=1, keepdims=True)
        sel = cm == m
        pos = jnp.min(jnp.where(sel, iota, jnp.int32(1 << 20)),
                      axis=1, keepdims=True)
        cm_scr[...] = jnp.where(iota == pos, NEG2, cm)
        return jnp.where(lanek == k, pos, cid_acc)

    cid = lax.fori_loop(0, K, body, jnp.zeros((BB, K), jnp.int32))
    brow = lax.broadcasted_iota(jnp.int32, (BB, K), 0)
    sidx = (bi * (NT * BB * CPT) + (cid >> 4) * (BB * CPT)
            + brow * CPT + (cid & 15))
    sidx_ref[...] = sidx[None]
    cid_ref[...] = cid[None]


def _k4_body(cand_ref, cid_ref, gidx_ref, w_ref, cand_scr, gmap_scr):
    cand = cand_ref[0]
    cid = cid_ref[0]
    gmap = (cid[:, :, None] * 128
            + lax.broadcasted_iota(jnp.int32, (BB, K, 128), 2))
    cand_scr[...] = cand.reshape(BB, K * 128)
    gmap_scr[...] = gmap.reshape(BB, K * 128)
    lanek = lax.broadcasted_iota(jnp.int32, (BB, K), 1)

    def body(k, acc):
        v_acc, g_acc = acc
        cv = cand_scr[...]
        gm = gmap_scr[...]
        m = jnp.max(cv, axis=1, keepdims=True)
        sel = cv == m
        gi = jnp.min(jnp.where(sel, gm, jnp.int32(1 << 30)),
                     axis=1, keepdims=True)
        cand_scr[...] = jnp.where(sel & (gm == gi), NEG2, cv)
        v_acc = jnp.where(lanek == k, m, v_acc)
        g_acc = jnp.where(lanek == k, gi, g_acc)
        return (v_acc, g_acc)

    v, gidx = lax.fori_loop(
        0, K, body,
        (jnp.zeros((BB, K), jnp.float32), jnp.zeros((BB, K), jnp.int32)))
    r = jnp.maximum(v, 0.0)
    ex = jnp.exp(r - jnp.max(r, axis=1, keepdims=True))
    w = ex / jnp.sum(ex, axis=1, keepdims=True)
    gidx_ref[...] = gidx[None]
    w_ref[...] = w[None]


def _k6_body(g_ref, w_ref, o_ref):
    g = g_ref[0]
    w = w_ref[0]
    acc = g[:, 0, :] * w[:, 0][:, None]
    for k in range(1, K):
        acc = acc + g[:, k, :] * w[:, k][:, None]
    o_ref[...] = acc


def _sc_gather_rows(table, idx):
    n = idx.shape[0]
    per_w = n // 32
    steps = per_w // 128
    mesh = plsc.VectorSubcoreMesh(core_axis_name="c", subcore_axis_name="s")

    @functools.partial(
        pl.kernel,
        out_type=jax.ShapeDtypeStruct((n, 128), jnp.float32),
        mesh=mesh,
        scratch_types=[
            pltpu.VMEM((128,), jnp.int32),
            pltpu.VMEM((128, 128), jnp.float32),
            pltpu.SemaphoreType.DMA,
        ],
    )
    def gather(table_hbm, idx_hbm, out_hbm, idx_v, rows_v, sem):
        wid = lax.axis_index("s") * 2 + lax.axis_index("c")
        base0 = pl.multiple_of(wid * per_w, 128)

        def body(j, carry):
            base = pl.multiple_of(base0 + j * 128, 128)
            pltpu.sync_copy(idx_hbm.at[pl.ds(base, 128)], idx_v)
            pltpu.async_copy(table_hbm.at[idx_v], rows_v, sem).wait()
            pltpu.sync_copy(rows_v, out_hbm.at[pl.ds(base, 128)])
            return carry

        lax.fori_loop(0, steps, body, 0)

    return gather(table, idx)


def kernel(encoder_outs, mem_matrix):
    f32 = jnp.float32
    inv_nb = pl.pallas_call(
        _nb_body,
        grid=(NT,),
        in_specs=[pl.BlockSpec((MB, D), lambda mj: (mj, 0))],
        out_specs=pl.BlockSpec((1, 1, MB), lambda mj: (mj, 0, 0)),
        out_shape=jax.ShapeDtypeStruct((NT, 1, MB), f32),
    )(mem_matrix)

    sim, cm = pl.pallas_call(
        _k1_body,
        grid=(NB, NT),
        in_specs=[
            pl.BlockSpec((BB, D), lambda bi, mj: (bi, 0)),
            pl.BlockSpec((MB, D), lambda bi, mj: (mj, 0)),
            pl.BlockSpec((1, 1, MB), lambda bi, mj: (mj, 0, 0)),
        ],
        out_specs=[
            pl.BlockSpec((1, 1, BB, CPT, 128), lambda bi, mj: (bi, mj, 0, 0, 0)),
            pl.BlockSpec((1, 1, BB, CPT), lambda bi, mj: (bi, mj, 0, 0)),
        ],
        out_shape=[
            jax.ShapeDtypeStruct((NB, NT, BB, CPT, 128), f32),
            jax.ShapeDtypeStruct((NB, NT, BB, CPT), f32),
        ],
        compiler_params=pltpu.CompilerParams(
            dimension_semantics=("arbitrary", "arbitrary")),
    )(encoder_outs, mem_matrix, inv_nb)

    cmt = cm.transpose(0, 2, 1, 3).reshape(NB, BB, C)

    sidx, cid = pl.pallas_call(
        _k2_body,
        grid=(NB,),
        in_specs=[pl.BlockSpec((1, BB, C), lambda bi: (bi, 0, 0))],
        out_specs=[
            pl.BlockSpec((1, BB, K), lambda bi: (bi, 0, 0)),
            pl.BlockSpec((1, BB, K), lambda bi: (bi, 0, 0)),
        ],
        out_shape=[
            jax.ShapeDtypeStruct((NB, BB, K), jnp.int32),
            jax.ShapeDtypeStruct((NB, BB, K), jnp.int32),
        ],
        scratch_shapes=[pltpu.VMEM((BB, C), jnp.float32)],
    )(cmt)

    sim_flat = sim.reshape(NB * NT * BB * CPT, 128)
    cand_flat = _sc_gather_rows(sim_flat, sidx.reshape(-1))
    cand = cand_flat.reshape(NB, BB, K, 128)

    gidx, w = pl.pallas_call(
        _k4_body,
        grid=(NB,),
        in_specs=[
            pl.BlockSpec((1, BB, K, 128), lambda bi: (bi, 0, 0, 0)),
            pl.BlockSpec((1, BB, K), lambda bi: (bi, 0, 0)),
        ],
        out_specs=[
            pl.BlockSpec((1, BB, K), lambda bi: (bi, 0, 0)),
            pl.BlockSpec((1, BB, K), lambda bi: (bi, 0, 0)),
        ],
        out_shape=[
            jax.ShapeDtypeStruct((NB, BB, K), jnp.int32),
            jax.ShapeDtypeStruct((NB, BB, K), f32),
        ],
        scratch_shapes=[
            pltpu.VMEM((BB, K * 128), jnp.float32),
            pltpu.VMEM((BB, K * 128), jnp.int32),
        ],
    )(cand, cid)

    g_flat = _sc_gather_rows(mem_matrix, gidx.reshape(-1))
    g = g_flat.reshape(NB, BB, K, 128)

    out = pl.pallas_call(
        _k6_body,
        grid=(NB,),
        in_specs=[
            pl.BlockSpec((1, BB, K, 128), lambda bi: (bi, 0, 0, 0)),
            pl.BlockSpec((1, BB, K), lambda bi: (bi, 0, 0)),
        ],
        out_specs=pl.BlockSpec((BB, D), lambda bi: (bi, 0)),
        out_shape=jax.ShapeDtypeStruct((B, D), f32),
    )(g, w)
    return out
